# baseline (speedup 1.0000x reference)
import functools

import jax
import jax.numpy as jnp
from jax import lax
from jax.experimental import pallas as pl
from jax.experimental.pallas import tpu as pltpu
from jax.experimental.pallas import tpu_sc as plsc

N = 10000
D = 128
E = 320000

NC = 2
NS = 16
NW = NC * NS
C = 128
HB = 40
NH = 2
CHUNKS = NH * HB
EPW = CHUNKS * C
E_PAD = NW * EPW

N_PAD = 10112
RPT = N_PAD // NS

_mesh = plsc.VectorSubcoreMesh(
    core_axis_name="c", subcore_axis_name="s", num_cores=NC, num_subcores=NS
)


@functools.partial(
    pl.kernel,
    out_type=jax.ShapeDtypeStruct((NC * N_PAD,), jnp.float32),
    mesh=_mesh,
    scratch_types=[
        pltpu.VMEM((C,), jnp.int32),
        pltpu.VMEM((C,), jnp.float32),
        pltpu.VMEM((RPT,), jnp.float32),
        pltpu.VMEM_SHARED((N_PAD,), jnp.float32),
    ],
)
def _sc_degree(dst_hbm, ones_hbm, zeros_hbm, out, dst_v, ones_v, stage_v, acc_sh):
    cid = lax.axis_index("c")
    sid = lax.axis_index("s")
    wid = sid * NC + cid
    r0 = sid * RPT
    pltpu.sync_copy(zeros_hbm.at[pl.ds(r0, RPT)], stage_v)
    pltpu.sync_copy(stage_v, acc_sh.at[pl.ds(r0, RPT)])
    pltpu.sync_copy(ones_hbm, ones_v)
    plsc.subcore_barrier()
    base0 = wid * EPW

    def chunk(j, carry):
        base = base0 + j * C
        pltpu.sync_copy(dst_hbm.at[pl.ds(base, C)], dst_v)
        pltpu.sync_copy(ones_v, acc_sh.at[dst_v], add=True)
        return carry

    lax.fori_loop(0, CHUNKS, chunk, 0)
    plsc.subcore_barrier()
    pltpu.sync_copy(acc_sh.at[pl.ds(r0, RPT)], stage_v)
    pltpu.sync_copy(stage_v, out.at[pl.ds(cid * N_PAD + r0, RPT)])


@functools.partial(
    pl.kernel,
    out_type=jax.ShapeDtypeStruct((NC, N_PAD, D), jnp.float32),
    mesh=_mesh,
    scratch_types=[
        pltpu.VMEM((HB, C), jnp.int32),
        pltpu.VMEM((HB, C), jnp.int32),
        pltpu.VMEM((C, D), jnp.float32),
        pltpu.VMEM((C, D), jnp.float32),
        pltpu.VMEM_SHARED((N_PAD, D), jnp.float32),
        pltpu.SemaphoreType.DMA,
        pltpu.SemaphoreType.DMA,
    ],
)
def _sc_aggregate(h_hbm, src_hbm, dst_hbm, zeros_hbm, out,
                  src_blk, dst_blk, rows_a, rows_b, acc_sh, sem_a, sem_b):
    cid = lax.axis_index("c")
    sid = lax.axis_index("s")
    wid = sid * NC + cid
    r0 = sid * RPT
    pltpu.sync_copy(zeros_hbm.at[pl.ds(r0, RPT)], acc_sh.at[pl.ds(r0, RPT)])

    def start_gather(j, rows_v, sem):
        pltpu.async_copy(h_hbm.at[src_blk.at[j]], rows_v, sem)

    def wait_gather(j, rows_v, sem):
        pltpu.make_async_copy(h_hbm.at[src_blk.at[j]], rows_v, sem).wait()

    def scatter(j, rows_v):
        pltpu.sync_copy(rows_v, acc_sh.at[dst_blk.at[j]], add=True)

    for h in range(NH):
        pltpu.sync_copy(src_hbm.at[wid, h], src_blk)
        pltpu.sync_copy(dst_hbm.at[wid, h], dst_blk)
        start_gather(0, rows_a, sem_a)

        def body(m, carry):
            ja = 2 * m
            start_gather(ja + 1, rows_b, sem_b)
            wait_gather(ja, rows_a, sem_a)
            scatter(ja, rows_a)
            start_gather(ja + 2, rows_a, sem_a)
            wait_gather(ja + 1, rows_b, sem_b)
            scatter(ja + 1, rows_b)
            return carry

        lax.fori_loop(0, HB // 2 - 1, body, 0)
        start_gather(HB - 1, rows_b, sem_b)
        wait_gather(HB - 2, rows_a, sem_a)
        scatter(HB - 2, rows_a)
        wait_gather(HB - 1, rows_b, sem_b)
        scatter(HB - 1, rows_b)

    plsc.subcore_barrier()
    pltpu.sync_copy(acc_sh.at[pl.ds(r0, RPT)], out.at[cid, pl.ds(r0, RPT)])


R = 1000


def _tc_first_body(x_ref, w_ref, d0_ref, d1_ref, h_ref, dis_ref):
    deg = d0_ref[...] + d1_ref[...] + 1.0
    dis = lax.rsqrt(deg)
    h = jnp.dot(x_ref[...], w_ref[...], preferred_element_type=jnp.float32)
    h_ref[...] = h * dis
    dis_ref[...] = dis


def _tc_mid_body(a0_ref, a1_ref, h1_ref, dis_ref, b1_ref, w2_ref, h2_ref):
    dis = dis_ref[...]
    z = (a0_ref[...] + a1_ref[...] + h1_ref[...]) * dis + b1_ref[...]
    z = jnp.maximum(z, 0.0)
    h2_ref[...] = jnp.dot(z, w2_ref[...], preferred_element_type=jnp.float32) * dis


def _tc_last_body(c0_ref, c1_ref, h2_ref, dis_ref, b2_ref, out_ref):
    out_ref[...] = (c0_ref[...] + c1_ref[...] + h2_ref[...]) * dis_ref[...] \
        + b2_ref[...]


_row_blk = pl.BlockSpec((R, D), lambda i: (i, 0))
_dis_blk = pl.BlockSpec((R, 1), lambda i: (i, 0))
_mat_blk = pl.BlockSpec((D, D), lambda i: (0, 0))
_bias_blk = pl.BlockSpec((1, D), lambda i: (0, 0))

_tc_first = pl.pallas_call(
    _tc_first_body,
    grid=(N // R,),
    in_specs=[_row_blk, _mat_blk, _dis_blk, _dis_blk],
    out_specs=(_row_blk, _dis_blk),
    out_shape=(
        jax.ShapeDtypeStruct((N, D), jnp.float32),
        jax.ShapeDtypeStruct((N, 1), jnp.float32),
    ),
)

_tc_mid = pl.pallas_call(
    _tc_mid_body,
    grid=(N // R,),
    in_specs=[_row_blk, _row_blk, _row_blk, _dis_blk, _bias_blk, _mat_blk],
    out_specs=_row_blk,
    out_shape=jax.ShapeDtypeStruct((N, D), jnp.float32),
)

_tc_last = pl.pallas_call(
    _tc_last_body,
    grid=(N // R,),
    in_specs=[_row_blk, _row_blk, _row_blk, _dis_blk, _bias_blk],
    out_specs=_row_blk,
    out_shape=jax.ShapeDtypeStruct((N, D), jnp.float32),
)


def kernel(x, edge_index, W1, b1, W2, b2):
    src = edge_index[0].astype(jnp.int32)
    dst = edge_index[1].astype(jnp.int32)
    pad = E_PAD - E
    src_p = jnp.concatenate([src, jnp.zeros((pad,), jnp.int32)])
    dst_p = jnp.concatenate([dst, jnp.full((pad,), N, jnp.int32)])
    src4 = src_p.reshape(NW, NH, HB, C)
    dst4 = dst_p.reshape(NW, NH, HB, C)

    ones1 = jnp.ones((C,), jnp.float32)
    zeros1 = jnp.zeros((N_PAD,), jnp.float32)
    zerosD = jnp.zeros((N_PAD, D), jnp.float32)

    deg = _sc_degree(dst_p, ones1, zeros1).reshape(NC, N_PAD)
    h1, dis = _tc_first(x, W1, deg[0, :N].reshape(N, 1),
                        deg[1, :N].reshape(N, 1))
    a = _sc_aggregate(h1, src4, dst4, zerosD)
    h2 = _tc_mid(a[0, :N], a[1, :N], h1, dis, b1.reshape(1, D), W2)
    c = _sc_aggregate(h2, src4, dst4, zerosD)
    return _tc_last(c[0, :N], c[1, :N], h2, dis, b2.reshape(1, D))


# speedup vs baseline: 1.0770x; 1.0770x over previous
import functools

import jax
import jax.numpy as jnp
from jax import lax
from jax.experimental import pallas as pl
from jax.experimental.pallas import tpu as pltpu
from jax.experimental.pallas import tpu_sc as plsc

N = 10000
D = 128
E = 320000

NC = 2
NS = 16
NW = NC * NS
C = 128
HB = 32
NBLK = 80
B0 = 4
B1 = 1
E_PAD = NBLK * HB * C
CHUNKS = 80
EPW = CHUNKS * C

N_PAD = 10112
RPT = N_PAD // NS

_mesh = plsc.VectorSubcoreMesh(
    core_axis_name="c", subcore_axis_name="s", num_cores=NC, num_subcores=NS
)


@functools.partial(
    pl.kernel,
    out_type=jax.ShapeDtypeStruct((NC * N_PAD,), jnp.float32),
    mesh=_mesh,
    scratch_types=[
        pltpu.VMEM((C,), jnp.int32),
        pltpu.VMEM((C,), jnp.float32),
        pltpu.VMEM((RPT,), jnp.float32),
        pltpu.VMEM_SHARED((N_PAD,), jnp.float32),
    ],
)
def _sc_degree(dst_hbm, ones_hbm, zeros_hbm, out, dst_v, ones_v, stage_v, acc_sh):
    cid = lax.axis_index("c")
    sid = lax.axis_index("s")
    wid = sid * NC + cid
    r0 = sid * RPT
    pltpu.sync_copy(zeros_hbm.at[pl.ds(r0, RPT)], stage_v)
    pltpu.sync_copy(stage_v, acc_sh.at[pl.ds(r0, RPT)])
    pltpu.sync_copy(ones_hbm, ones_v)
    plsc.subcore_barrier()
    base0 = wid * EPW

    def chunk(j, carry):
        base = base0 + j * C
        pltpu.sync_copy(dst_hbm.at[pl.ds(base, C)], dst_v)
        pltpu.sync_copy(ones_v, acc_sh.at[dst_v], add=True)
        return carry

    lax.fori_loop(0, CHUNKS, chunk, 0)
    plsc.subcore_barrier()
    pltpu.sync_copy(acc_sh.at[pl.ds(r0, RPT)], stage_v)
    pltpu.sync_copy(stage_v, out.at[pl.ds(cid * N_PAD + r0, RPT)])


@functools.partial(
    pl.kernel,
    out_type=jax.ShapeDtypeStruct((NC, N_PAD, D), jnp.float32),
    mesh=_mesh,
    scratch_types=[
        pltpu.VMEM((HB, C), jnp.int32),
        pltpu.VMEM((HB, C), jnp.int32),
        pltpu.VMEM((C, D), jnp.float32),
        pltpu.VMEM((C, D), jnp.float32),
        pltpu.VMEM_SHARED((N_PAD, D), jnp.float32),
        pltpu.SemaphoreType.DMA,
        pltpu.SemaphoreType.DMA,
    ],
)
def _sc_aggregate(h_hbm, src_hbm, dst_hbm, zeros_hbm, out,
                  src_blk, dst_blk, rows_a, rows_b, acc_sh, sem_a, sem_b):
    cid = lax.axis_index("c")
    sid = lax.axis_index("s")
    r0 = sid * RPT
    pltpu.sync_copy(zeros_hbm.at[pl.ds(r0, RPT)], acc_sh.at[pl.ds(r0, RPT)])

    def start_gather(j, rows_v, sem):
        pltpu.async_copy(h_hbm.at[src_blk.at[j]], rows_v, sem)

    def wait_gather(j, rows_v, sem):
        pltpu.make_async_copy(h_hbm.at[src_blk.at[j]], rows_v, sem).wait()

    def scatter(j, rows_v):
        pltpu.sync_copy(rows_v, acc_sh.at[dst_blk.at[j]], add=True)

    nblk = jnp.where(cid == 0, B0, B1)
    gbase = jnp.where(cid == 0, sid * B0, 16 * B0 + sid * B1)

    def blk_body(b, carry):
        g = gbase + b
        pltpu.sync_copy(src_hbm.at[g], src_blk)
        pltpu.sync_copy(dst_hbm.at[g], dst_blk)
        start_gather(0, rows_a, sem_a)

        def body(m, carry2):
            ja = 2 * m
            start_gather(ja + 1, rows_b, sem_b)
            wait_gather(ja, rows_a, sem_a)
            scatter(ja, rows_a)
            start_gather(ja + 2, rows_a, sem_a)
            wait_gather(ja + 1, rows_b, sem_b)
            scatter(ja + 1, rows_b)
            return carry2

        lax.fori_loop(0, HB // 2 - 1, body, 0)
        start_gather(HB - 1, rows_b, sem_b)
        wait_gather(HB - 2, rows_a, sem_a)
        scatter(HB - 2, rows_a)
        wait_gather(HB - 1, rows_b, sem_b)
        scatter(HB - 1, rows_b)
        return carry

    lax.fori_loop(0, nblk, blk_body, 0)
    plsc.subcore_barrier()
    pltpu.sync_copy(acc_sh.at[pl.ds(r0, RPT)], out.at[cid, pl.ds(r0, RPT)])


R = 1000


def _tc_first_body(x_ref, w_ref, d0_ref, d1_ref, h_ref, dis_ref):
    deg = d0_ref[...] + d1_ref[...] + 1.0
    dis = lax.rsqrt(deg)
    h = jnp.dot(x_ref[...], w_ref[...], preferred_element_type=jnp.float32)
    h_ref[...] = h * dis
    dis_ref[...] = dis


def _tc_mid_body(a0_ref, a1_ref, h1_ref, dis_ref, b1_ref, w2_ref, h2_ref):
    dis = dis_ref[...]
    z = (a0_ref[...] + a1_ref[...] + h1_ref[...]) * dis + b1_ref[...]
    z = jnp.maximum(z, 0.0)
    h2_ref[...] = jnp.dot(z, w2_ref[...], preferred_element_type=jnp.float32) * dis


def _tc_last_body(c0_ref, c1_ref, h2_ref, dis_ref, b2_ref, out_ref):
    out_ref[...] = (c0_ref[...] + c1_ref[...] + h2_ref[...]) * dis_ref[...] \
        + b2_ref[...]


_row_blk = pl.BlockSpec((R, D), lambda i: (i, 0))
_dis_blk = pl.BlockSpec((R, 1), lambda i: (i, 0))
_mat_blk = pl.BlockSpec((D, D), lambda i: (0, 0))
_bias_blk = pl.BlockSpec((1, D), lambda i: (0, 0))

_tc_first = pl.pallas_call(
    _tc_first_body,
    grid=(N // R,),
    in_specs=[_row_blk, _mat_blk, _dis_blk, _dis_blk],
    out_specs=(_row_blk, _dis_blk),
    out_shape=(
        jax.ShapeDtypeStruct((N, D), jnp.float32),
        jax.ShapeDtypeStruct((N, 1), jnp.float32),
    ),
)

_tc_mid = pl.pallas_call(
    _tc_mid_body,
    grid=(N // R,),
    in_specs=[_row_blk, _row_blk, _row_blk, _dis_blk, _bias_blk, _mat_blk],
    out_specs=_row_blk,
    out_shape=jax.ShapeDtypeStruct((N, D), jnp.float32),
)

_tc_last = pl.pallas_call(
    _tc_last_body,
    grid=(N // R,),
    in_specs=[_row_blk, _row_blk, _row_blk, _dis_blk, _bias_blk],
    out_specs=_row_blk,
    out_shape=jax.ShapeDtypeStruct((N, D), jnp.float32),
)


def kernel(x, edge_index, W1, b1, W2, b2):
    src = edge_index[0].astype(jnp.int32)
    dst = edge_index[1].astype(jnp.int32)
    pad = E_PAD - E
    src_p = jnp.concatenate([src, jnp.zeros((pad,), jnp.int32)])
    dst_p = jnp.concatenate([dst, jnp.full((pad,), N, jnp.int32)])
    src4 = src_p.reshape(NBLK, HB, C)
    dst4 = dst_p.reshape(NBLK, HB, C)

    ones1 = jnp.ones((C,), jnp.float32)
    zeros1 = jnp.zeros((N_PAD,), jnp.float32)
    zerosD = jnp.zeros((N_PAD, D), jnp.float32)

    deg = _sc_degree(dst_p, ones1, zeros1).reshape(NC, N_PAD)
    h1, dis = _tc_first(x, W1, deg[0, :N].reshape(N, 1),
                        deg[1, :N].reshape(N, 1))
    a = _sc_aggregate(h1, src4, dst4, zerosD)
    h2 = _tc_mid(a[0, :N], a[1, :N], h1, dis, b1.reshape(1, D), W2)
    c = _sc_aggregate(h2, src4, dst4, zerosD)
    return _tc_last(c[0, :N], c[1, :N], h2, dis, b2.reshape(1, D))


# speedup vs baseline: 2.9185x; 2.7100x over previous
import functools

import jax
import jax.numpy as jnp
from jax import lax
from jax.experimental import pallas as pl
from jax.experimental.pallas import tpu as pltpu
from jax.experimental.pallas import tpu_sc as plsc

N = 10000
D = 128
E = 320000

NC = 2
NS = 16
NW = NC * NS
C = 128
HB = 16
BPW = 5
NBLK = NW * BPW
E_PAD = NBLK * HB * C
CHUNKS = 80
EPW = CHUNKS * C

N_PAD = 10112
RPT = N_PAD // NS

_mesh = plsc.VectorSubcoreMesh(
    core_axis_name="c", subcore_axis_name="s", num_cores=NC, num_subcores=NS
)


@functools.partial(
    pl.kernel,
    out_type=jax.ShapeDtypeStruct((NC * N_PAD,), jnp.float32),
    mesh=_mesh,
    scratch_types=[
        pltpu.VMEM((C,), jnp.int32),
        pltpu.VMEM((C,), jnp.float32),
        pltpu.VMEM((RPT,), jnp.float32),
        pltpu.VMEM_SHARED((N_PAD,), jnp.float32),
    ],
)
def _sc_degree(dst_hbm, ones_hbm, zeros_hbm, out, dst_v, ones_v, stage_v, acc_sh):
    cid = lax.axis_index("c")
    sid = lax.axis_index("s")
    wid = sid * NC + cid
    r0 = sid * RPT
    pltpu.sync_copy(zeros_hbm.at[pl.ds(r0, RPT)], stage_v)
    pltpu.sync_copy(stage_v, acc_sh.at[pl.ds(r0, RPT)])
    pltpu.sync_copy(ones_hbm, ones_v)
    plsc.subcore_barrier()
    base0 = wid * EPW

    def chunk(j, carry):
        base = base0 + j * C
        pltpu.sync_copy(dst_hbm.at[pl.ds(base, C)], dst_v)
        pltpu.sync_copy(ones_v, acc_sh.at[dst_v], add=True)
        return carry

    lax.fori_loop(0, CHUNKS, chunk, 0)
    plsc.subcore_barrier()
    pltpu.sync_copy(acc_sh.at[pl.ds(r0, RPT)], stage_v)
    pltpu.sync_copy(stage_v, out.at[pl.ds(cid * N_PAD + r0, RPT)])


@functools.partial(
    pl.kernel,
    out_type=jax.ShapeDtypeStruct((NC, N_PAD, D), jnp.float32),
    mesh=_mesh,
    scratch_types=[
        pltpu.VMEM((HB, C), jnp.int32),
        pltpu.VMEM((HB, C), jnp.int32),
        pltpu.VMEM((C, D), jnp.float32),
        pltpu.VMEM((C, D), jnp.float32),
        pltpu.VMEM_SHARED((N_PAD, D), jnp.float32),
        pltpu.SemaphoreType.DMA,
        pltpu.SemaphoreType.DMA,
    ],
)
def _sc_aggregate(h_hbm, src_hbm, dst_hbm, zeros_hbm, out,
                  src_blk, dst_blk, rows_a, rows_b, acc_sh, sem_a, sem_b):
    cid = lax.axis_index("c")
    sid = lax.axis_index("s")
    r0 = sid * RPT
    pltpu.sync_copy(zeros_hbm.at[pl.ds(r0, RPT)], acc_sh.at[pl.ds(r0, RPT)])
    plsc.subcore_barrier()

    def start_gather(j, rows_v, sem):
        pltpu.async_copy(h_hbm.at[src_blk.at[j]], rows_v, sem)

    def wait_gather(j, rows_v, sem):
        pltpu.make_async_copy(h_hbm.at[src_blk.at[j]], rows_v, sem).wait()

    def scatter(j, rows_v):
        pltpu.sync_copy(rows_v, acc_sh.at[dst_blk.at[j]], add=True)

    gbase = (sid * NC + cid) * BPW

    def blk_body(b, carry):
        g = gbase + b
        pltpu.sync_copy(src_hbm.at[g], src_blk)
        pltpu.sync_copy(dst_hbm.at[g], dst_blk)
        start_gather(0, rows_a, sem_a)

        def body(m, carry2):
            ja = 2 * m
            start_gather(ja + 1, rows_b, sem_b)
            wait_gather(ja, rows_a, sem_a)
            scatter(ja, rows_a)
            start_gather(ja + 2, rows_a, sem_a)
            wait_gather(ja + 1, rows_b, sem_b)
            scatter(ja + 1, rows_b)
            return carry2

        lax.fori_loop(0, HB // 2 - 1, body, 0)
        start_gather(HB - 1, rows_b, sem_b)
        wait_gather(HB - 2, rows_a, sem_a)
        scatter(HB - 2, rows_a)
        wait_gather(HB - 1, rows_b, sem_b)
        scatter(HB - 1, rows_b)
        return carry

    lax.fori_loop(0, BPW, blk_body, 0)
    plsc.subcore_barrier()
    pltpu.sync_copy(acc_sh.at[pl.ds(r0, RPT)], out.at[cid, pl.ds(r0, RPT)])


R = 1000


def _tc_first_body(x_ref, w_ref, d0_ref, d1_ref, h_ref, dis_ref):
    deg = d0_ref[...] + d1_ref[...] + 1.0
    dis = lax.rsqrt(deg)
    h = jnp.dot(x_ref[...], w_ref[...], preferred_element_type=jnp.float32)
    h_ref[...] = h * dis
    dis_ref[...] = dis


def _tc_mid_body(a0_ref, a1_ref, h1_ref, dis_ref, b1_ref, w2_ref, h2_ref):
    dis = dis_ref[...]
    z = (a0_ref[...] + a1_ref[...] + h1_ref[...]) * dis + b1_ref[...]
    z = jnp.maximum(z, 0.0)
    h2_ref[...] = jnp.dot(z, w2_ref[...], preferred_element_type=jnp.float32) * dis


def _tc_last_body(c0_ref, c1_ref, h2_ref, dis_ref, b2_ref, out_ref):
    out_ref[...] = (c0_ref[...] + c1_ref[...] + h2_ref[...]) * dis_ref[...] \
        + b2_ref[...]


_row_blk = pl.BlockSpec((R, D), lambda i: (i, 0))
_dis_blk = pl.BlockSpec((R, 1), lambda i: (i, 0))
_mat_blk = pl.BlockSpec((D, D), lambda i: (0, 0))
_bias_blk = pl.BlockSpec((1, D), lambda i: (0, 0))

_tc_first = pl.pallas_call(
    _tc_first_body,
    grid=(N // R,),
    in_specs=[_row_blk, _mat_blk, _dis_blk, _dis_blk],
    out_specs=(_row_blk, _dis_blk),
    out_shape=(
        jax.ShapeDtypeStruct((N, D), jnp.float32),
        jax.ShapeDtypeStruct((N, 1), jnp.float32),
    ),
)

_tc_mid = pl.pallas_call(
    _tc_mid_body,
    grid=(N // R,),
    in_specs=[_row_blk, _row_blk, _row_blk, _dis_blk, _bias_blk, _mat_blk],
    out_specs=_row_blk,
    out_shape=jax.ShapeDtypeStruct((N, D), jnp.float32),
)

_tc_last = pl.pallas_call(
    _tc_last_body,
    grid=(N // R,),
    in_specs=[_row_blk, _row_blk, _row_blk, _dis_blk, _bias_blk],
    out_specs=_row_blk,
    out_shape=jax.ShapeDtypeStruct((N, D), jnp.float32),
)


def kernel(x, edge_index, W1, b1, W2, b2):
    src = edge_index[0].astype(jnp.int32)
    dst = edge_index[1].astype(jnp.int32)
    pad = E_PAD - E
    pad_i = jnp.arange(pad, dtype=jnp.int32)
    src_p = jnp.concatenate([src, pad_i % N])
    dst_p = jnp.concatenate([dst, N + pad_i % (N_PAD - N)])
    src4 = src_p.reshape(NBLK, HB, C)
    dst4 = dst_p.reshape(NBLK, HB, C)

    ones1 = jnp.ones((C,), jnp.float32)
    zeros1 = jnp.zeros((N_PAD,), jnp.float32)
    zerosD = jnp.zeros((N_PAD, D), jnp.float32)

    deg = _sc_degree(dst_p, ones1, zeros1).reshape(NC, N_PAD)
    h1, dis = _tc_first(x, W1, deg[0, :N].reshape(N, 1),
                        deg[1, :N].reshape(N, 1))
    a = _sc_aggregate(h1, src4, dst4, zerosD)
    h2 = _tc_mid(a[0, :N], a[1, :N], h1, dis, b1.reshape(1, D), W2)
    c = _sc_aggregate(h2, src4, dst4, zerosD)
    return _tc_last(c[0, :N], c[1, :N], h2, dis, b2.reshape(1, D))


# speedup vs baseline: 3.3396x; 1.1443x over previous
import functools

import jax
import jax.numpy as jnp
from jax import lax
from jax.experimental import pallas as pl
from jax.experimental.pallas import tpu as pltpu
from jax.experimental.pallas import tpu_sc as plsc

N = 10000
D = 128
E = 320000

NC = 2
NS = 16
NW = NC * NS
C = 128
HB = 16
BPW = 5
NBLK = NW * BPW
E_PAD = NBLK * HB * C
CHUNKS = 80
EPW = CHUNKS * C

N_PAD = 10112
RPT = N_PAD // NS

_mesh = plsc.VectorSubcoreMesh(
    core_axis_name="c", subcore_axis_name="s", num_cores=NC, num_subcores=NS
)


@functools.partial(
    pl.kernel,
    out_type=jax.ShapeDtypeStruct((NC * N_PAD,), jnp.float32),
    mesh=_mesh,
    scratch_types=[
        pltpu.VMEM((HB, C), jnp.int32),
        pltpu.VMEM((C,), jnp.float32),
        pltpu.VMEM((RPT,), jnp.float32),
        pltpu.VMEM_SHARED((N_PAD,), jnp.float32),
    ],
)
def _sc_degree(dst_hbm, ones_hbm, zeros_hbm, out, dst_blk, ones_v, stage_v, acc_sh):
    cid = lax.axis_index("c")
    sid = lax.axis_index("s")
    r0 = sid * RPT
    pltpu.sync_copy(zeros_hbm.at[pl.ds(r0, RPT)], stage_v)
    pltpu.sync_copy(stage_v, acc_sh.at[pl.ds(r0, RPT)])
    pltpu.sync_copy(ones_hbm, ones_v)
    plsc.subcore_barrier()
    gbase = (sid * NC + cid) * BPW

    def blk_body(b, carry):
        pltpu.sync_copy(dst_hbm.at[gbase + b], dst_blk)

        def chunk(j, carry2):
            pltpu.sync_copy(ones_v, acc_sh.at[dst_blk.at[j]], add=True)
            return carry2

        lax.fori_loop(0, HB, chunk, 0)
        return carry

    lax.fori_loop(0, BPW, blk_body, 0)
    plsc.subcore_barrier()
    pltpu.sync_copy(acc_sh.at[pl.ds(r0, RPT)], stage_v)
    pltpu.sync_copy(stage_v, out.at[pl.ds(cid * N_PAD + r0, RPT)])


@functools.partial(
    pl.kernel,
    out_type=jax.ShapeDtypeStruct((NC, N_PAD, D), jnp.float32),
    mesh=_mesh,
    scratch_types=[
        pltpu.VMEM((HB, C), jnp.int32),
        pltpu.VMEM((HB, C), jnp.int32),
        pltpu.VMEM((C, D), jnp.float32),
        pltpu.VMEM((C, D), jnp.float32),
        pltpu.VMEM_SHARED((N_PAD, D), jnp.float32),
        pltpu.SemaphoreType.DMA,
        pltpu.SemaphoreType.DMA,
    ],
)
def _sc_aggregate(h_hbm, src_hbm, dst_hbm, zeros_hbm, out,
                  src_blk, dst_blk, rows_a, rows_b, acc_sh, sem_a, sem_b):
    cid = lax.axis_index("c")
    sid = lax.axis_index("s")
    r0 = sid * RPT
    pltpu.sync_copy(zeros_hbm.at[pl.ds(r0, RPT)], acc_sh.at[pl.ds(r0, RPT)])
    plsc.subcore_barrier()

    def start_gather(j, rows_v, sem):
        pltpu.async_copy(h_hbm.at[src_blk.at[j]], rows_v, sem)

    def wait_gather(j, rows_v, sem):
        pltpu.make_async_copy(h_hbm.at[src_blk.at[j]], rows_v, sem).wait()

    def scatter(j, rows_v):
        pltpu.sync_copy(rows_v, acc_sh.at[dst_blk.at[j]], add=True)

    gbase = (sid * NC + cid) * BPW

    def blk_body(b, carry):
        g = gbase + b
        pltpu.sync_copy(src_hbm.at[g], src_blk)
        pltpu.sync_copy(dst_hbm.at[g], dst_blk)
        start_gather(0, rows_a, sem_a)

        def body(m, carry2):
            ja = 2 * m
            start_gather(ja + 1, rows_b, sem_b)
            wait_gather(ja, rows_a, sem_a)
            scatter(ja, rows_a)
            start_gather(ja + 2, rows_a, sem_a)
            wait_gather(ja + 1, rows_b, sem_b)
            scatter(ja + 1, rows_b)
            return carry2

        lax.fori_loop(0, HB // 2 - 1, body, 0)
        start_gather(HB - 1, rows_b, sem_b)
        wait_gather(HB - 2, rows_a, sem_a)
        scatter(HB - 2, rows_a)
        wait_gather(HB - 1, rows_b, sem_b)
        scatter(HB - 1, rows_b)
        return carry

    lax.fori_loop(0, BPW, blk_body, 0)
    plsc.subcore_barrier()
    pltpu.sync_copy(acc_sh.at[pl.ds(r0, RPT)], out.at[cid, pl.ds(r0, RPT)])


R = 1000


def _tc_first_body(x_ref, w_ref, d0_ref, d1_ref, h_ref, dis_ref):
    deg = d0_ref[...] + d1_ref[...] + 1.0
    dis = lax.rsqrt(deg)
    h = jnp.dot(x_ref[...], w_ref[...], preferred_element_type=jnp.float32)
    h_ref[...] = h * dis
    dis_ref[...] = dis


def _tc_mid_body(a_ref, h1_ref, dis_ref, b1_ref, w2_ref, h2_ref):
    dis = dis_ref[...]
    z = (a_ref[0] + a_ref[1] + h1_ref[...]) * dis + b1_ref[...]
    z = jnp.maximum(z, 0.0)
    h2_ref[...] = jnp.dot(z, w2_ref[...], preferred_element_type=jnp.float32) * dis


def _tc_last_body(c_ref, h2_ref, dis_ref, b2_ref, out_ref):
    out_ref[...] = (c_ref[0] + c_ref[1] + h2_ref[...]) * dis_ref[...] \
        + b2_ref[...]


_row_blk = pl.BlockSpec((R, D), lambda i: (i, 0))
_acc_blk = pl.BlockSpec((NC, R, D), lambda i: (0, i, 0))
_dis_blk = pl.BlockSpec((R, 1), lambda i: (i, 0))
_mat_blk = pl.BlockSpec((D, D), lambda i: (0, 0))
_bias_blk = pl.BlockSpec((1, D), lambda i: (0, 0))

_tc_first = pl.pallas_call(
    _tc_first_body,
    grid=(N // R,),
    in_specs=[_row_blk, _mat_blk, _dis_blk, _dis_blk],
    out_specs=(_row_blk, _dis_blk),
    out_shape=(
        jax.ShapeDtypeStruct((N, D), jnp.float32),
        jax.ShapeDtypeStruct((N, 1), jnp.float32),
    ),
)

_tc_mid = pl.pallas_call(
    _tc_mid_body,
    grid=(N // R,),
    in_specs=[_acc_blk, _row_blk, _dis_blk, _bias_blk, _mat_blk],
    out_specs=_row_blk,
    out_shape=jax.ShapeDtypeStruct((N, D), jnp.float32),
)

_tc_last = pl.pallas_call(
    _tc_last_body,
    grid=(N // R,),
    in_specs=[_acc_blk, _row_blk, _dis_blk, _bias_blk],
    out_specs=_row_blk,
    out_shape=jax.ShapeDtypeStruct((N, D), jnp.float32),
)


def kernel(x, edge_index, W1, b1, W2, b2):
    src = edge_index[0].astype(jnp.int32)
    dst = edge_index[1].astype(jnp.int32)
    pad = E_PAD - E
    pad_i = jnp.arange(pad, dtype=jnp.int32)
    src_p = jnp.concatenate([src, pad_i % N])
    dst_p = jnp.concatenate([dst, N + pad_i % (N_PAD - N)])
    src4 = src_p.reshape(NBLK, HB, C)
    dst4 = dst_p.reshape(NBLK, HB, C)

    ones1 = jnp.ones((C,), jnp.float32)
    zeros1 = jnp.zeros((N_PAD,), jnp.float32)
    zerosD = jnp.zeros((N_PAD, D), jnp.float32)

    deg = _sc_degree(dst4, ones1, zeros1).reshape(NC, N_PAD)
    h1, dis = _tc_first(x, W1, deg[0, :N].reshape(N, 1),
                        deg[1, :N].reshape(N, 1))
    a = _sc_aggregate(h1, src4, dst4, zerosD)
    h2 = _tc_mid(a, h1, dis, b1.reshape(1, D), W2)
    c = _sc_aggregate(h2, src4, dst4, zerosD)
    return _tc_last(c, h2, dis, b2.reshape(1, D))
